# baseline (speedup 1.0000x reference)
import jax
import jax.numpy as jnp
from jax.experimental import pallas as pl
from jax.experimental.pallas import tpu as pltpu


def _mm(a, b):
    return jax.lax.dot_general(
        a, b, (((1,), (0,)), ((), ())),
        precision=jax.lax.Precision.DEFAULT,
        preferred_element_type=jnp.float32)


def _body(x_ref, w1_ref, adj_ref, b1_ref, w2_ref, b2_ref, o_ref,
          s1_ref, s2_ref):
    i = pl.program_id(0)
    nsteps = pl.num_programs(0)
    p = nsteps // 2
    bi = adj_ref.shape[0]

    @pl.when(i == 0)
    def _():
        s1_ref[...] = _mm(x_ref[...], w1_ref[...])

    @pl.when(i < p)
    def _():
        t = _mm(adj_ref[...], s1_ref[...])
        h = jnp.maximum(t + b1_ref[...], 0.0)
        s2_ref[pl.ds(i * bi, bi), :] = _mm(h, w2_ref[...])

    @pl.when(i >= p)
    def _():
        t = _mm(adj_ref[...], s2_ref[...])
        logits = t + b2_ref[...]
        m = jnp.max(logits, axis=1, keepdims=True)
        e = jnp.exp(logits - m)
        lse = m + jnp.log(jnp.sum(e, axis=1, keepdims=True))
        o_ref[...] = logits - lse


def kernel(raw_x, adj, W1, b1, W2, b2):
    n, nfeat = raw_x.shape
    nhid = W1.shape[1]
    ncls = W2.shape[1]
    b1r = b1.reshape(1, nhid)
    b2r = b2.reshape(1, ncls)

    BI = min(200, n)
    P = n // BI

    adj_map = lambda i: (jax.lax.rem(i, P), 0)
    const = lambda i: (0, 0)

    pred = pl.pallas_call(
        _body,
        grid=(2 * P,),
        in_specs=[
            pl.BlockSpec((n, nfeat), const),
            pl.BlockSpec((nfeat, nhid), const),
            pl.BlockSpec((BI, n), adj_map),
            pl.BlockSpec((1, nhid), const),
            pl.BlockSpec((nhid, ncls), const),
            pl.BlockSpec((1, ncls), const),
        ],
        out_specs=pl.BlockSpec(
            (BI, ncls), lambda i: (jnp.where(i < P, 0, i - P), 0)),
        out_shape=jax.ShapeDtypeStruct((n, ncls), jnp.float32),
        scratch_shapes=[
            pltpu.VMEM((n, nhid), jnp.float32),
            pltpu.VMEM((n, ncls), jnp.float32),
        ],
        compiler_params=pltpu.CompilerParams(
            dimension_semantics=("arbitrary",)),
    )(raw_x, W1, adj, b1r, W2, b2r)
    return pred


# speedup vs baseline: 1.0350x; 1.0350x over previous
import jax
import jax.numpy as jnp
from jax.experimental import pallas as pl
from jax.experimental.pallas import tpu as pltpu


def _mm(a, b):
    return jax.lax.dot_general(
        a, b, (((1,), (0,)), ((), ())),
        precision=jax.lax.Precision.DEFAULT,
        preferred_element_type=jnp.float32)


def _body(x_ref, w1_ref, adjl_ref, adjr_ref, b1_ref, w2_ref, b2_ref, o_ref,
          s1_ref, s2_ref):
    i = pl.program_id(0)
    nsteps = pl.num_programs(0)
    p = nsteps // 2
    hb = adjl_ref.shape[0]

    @pl.when(i == 0)
    def _():
        s1_ref[...] = _mm(x_ref[...], w1_ref[...])

    @pl.when(i < p)
    def _():
        tt = _mm(adjl_ref[...], s1_ref[...])
        tb = _mm(adjr_ref[...], s1_ref[...])
        ht = jnp.maximum(tt + b1_ref[...], 0.0)
        hb_ = jnp.maximum(tb + b1_ref[...], 0.0)
        s2_ref[pl.ds(2 * i * hb, hb), :] = _mm(ht, w2_ref[...])
        s2_ref[pl.ds((2 * i + 1) * hb, hb), :] = _mm(hb_, w2_ref[...])

    @pl.when(i >= p)
    def _():
        tt = _mm(adjl_ref[...], s2_ref[...])
        tb = _mm(adjr_ref[...], s2_ref[...])
        t = jnp.concatenate([tt, tb], axis=0)
        logits = t + b2_ref[...]
        m = jnp.max(logits, axis=1, keepdims=True)
        e = jnp.exp(logits - m)
        lse = m + jnp.log(jnp.sum(e, axis=1, keepdims=True))
        o_ref[...] = logits - lse


def kernel(raw_x, adj, W1, b1, W2, b2):
    n, nfeat = raw_x.shape
    nhid = W1.shape[1]
    ncls = W2.shape[1]
    b1r = b1.reshape(1, nhid)
    b2r = b2.reshape(1, ncls)

    BI = min(400, n)
    P = n // BI
    HB = BI // 2

    adjl_map = lambda i: (2 * jax.lax.rem(i, P), 0)
    adjr_map = lambda i: (2 * jax.lax.rem(i, P) + 1, 0)
    const = lambda i: (0, 0)

    pred = pl.pallas_call(
        _body,
        grid=(2 * P,),
        in_specs=[
            pl.BlockSpec((n, nfeat), const),
            pl.BlockSpec((nfeat, nhid), const),
            pl.BlockSpec((HB, n), adjl_map),
            pl.BlockSpec((HB, n), adjr_map),
            pl.BlockSpec((1, nhid), const),
            pl.BlockSpec((nhid, ncls), const),
            pl.BlockSpec((1, ncls), const),
        ],
        out_specs=pl.BlockSpec(
            (BI, ncls), lambda i: (jnp.where(i < P, 0, i - P), 0)),
        out_shape=jax.ShapeDtypeStruct((n, ncls), jnp.float32),
        scratch_shapes=[
            pltpu.VMEM((n, nhid), jnp.float32),
            pltpu.VMEM((n, ncls), jnp.float32),
        ],
        compiler_params=pltpu.CompilerParams(
            dimension_semantics=("arbitrary",)),
    )(raw_x, W1, adj, adj, b1r, W2, b2r)
    return pred


# speedup vs baseline: 1.0472x; 1.0119x over previous
import jax
import jax.numpy as jnp
from jax import lax
from jax.experimental import pallas as pl
from jax.experimental.pallas import tpu as pltpu

_NBUF = 4
_BI = 200


def _mm(a, b):
    return jax.lax.dot_general(
        a, b, (((1,), (0,)), ((), ())),
        precision=jax.lax.Precision.DEFAULT,
        preferred_element_type=jnp.float32)


def _body(x_ref, w1_ref, adj_ref, b1_ref, w2_ref, b2_ref, o_ref,
          b0, b1v, b2v, b3, ob0, ob1, s1_ref, s2_ref,
          sem0, sem1, sem2, sem3, osem0, osem1):
    n = adj_ref.shape[0]
    nstrips = n // _BI
    total = 2 * nstrips
    bufs = [b0, b1v, b2v, b3]
    sems = [sem0, sem1, sem2, sem3]
    obufs = [ob0, ob1]
    osems = [osem0, osem1]

    def _strip_copy(s, b):
        r = lax.rem(s, nstrips) * _BI
        return pltpu.make_async_copy(
            adj_ref.at[pl.ds(r, _BI), :], bufs[b], sems[b])

    def _out_copy(s, ob):
        r = lax.rem(s, nstrips) * _BI
        return pltpu.make_async_copy(
            obufs[ob], o_ref.at[pl.ds(r, _BI), :], osems[ob])

    for b in range(_NBUF):
        _strip_copy(jnp.int32(b), b).start()

    s1_ref[...] = _mm(x_ref[...], w1_ref[...])

    def _outer(g, carry):
        for b in range(_NBUF):
            s = _NBUF * g + b
            ob = b % 2
            _strip_copy(s, b).wait()

            @pl.when(s < nstrips)
            def _():
                t = _mm(bufs[b][...], s1_ref[...])
                h = jnp.maximum(t + b1_ref[...], 0.0)
                s2_ref[pl.ds(s * _BI, _BI), :] = _mm(h, w2_ref[...])

            @pl.when(s >= nstrips)
            def _():
                @pl.when(s >= nstrips + 2)
                def _():
                    _out_copy(s - 2, ob).wait()

                t = _mm(bufs[b][...], s2_ref[...])
                logits = t + b2_ref[...]
                m = jnp.max(logits, axis=1, keepdims=True)
                e = jnp.exp(logits - m)
                lse = m + jnp.log(jnp.sum(e, axis=1, keepdims=True))
                obufs[ob][...] = logits - lse
                _out_copy(s, ob).start()

            @pl.when(s + _NBUF < total)
            def _():
                _strip_copy(s + _NBUF, b).start()
        return carry

    lax.fori_loop(0, total // _NBUF, _outer, 0)

    _out_copy(jnp.int32(total - 2), 0).wait()
    _out_copy(jnp.int32(total - 1), 1).wait()


def kernel(raw_x, adj, W1, b1, W2, b2):
    n, nfeat = raw_x.shape
    nhid = W1.shape[1]
    ncls = W2.shape[1]
    b1r = b1.reshape(1, nhid)
    b2r = b2.reshape(1, ncls)

    vmem = pl.BlockSpec(memory_space=pltpu.MemorySpace.VMEM)
    hbm = pl.BlockSpec(memory_space=pltpu.MemorySpace.HBM)

    pred = pl.pallas_call(
        _body,
        in_specs=[vmem, vmem, hbm, vmem, vmem, vmem],
        out_specs=hbm,
        out_shape=jax.ShapeDtypeStruct((n, ncls), jnp.float32),
        scratch_shapes=(
            [pltpu.VMEM((_BI, n), jnp.float32) for _ in range(_NBUF)]
            + [pltpu.VMEM((_BI, ncls), jnp.float32) for _ in range(2)]
            + [pltpu.VMEM((n, nhid), jnp.float32),
               pltpu.VMEM((n, ncls), jnp.float32)]
            + [pltpu.SemaphoreType.DMA for _ in range(_NBUF + 2)]
        ),
    )(raw_x, W1, adj, b1r, W2, b2r)
    return pred
